# baseline (speedup 1.0000x reference)
import functools
import math

import jax
import jax.numpy as jnp
from jax import lax
from jax.experimental import pallas as pl
from jax.experimental.pallas import tpu as pltpu
from jax.experimental.pallas import tpu_sc as plsc

N_VOCAB = 100000
MAX_LENGTH = 4096
OUT_DIM = 128


@functools.cache
def _build(batch, seq, dim):
    info = plsc.get_sparse_core_info()
    nc, ns, lanes = info.num_cores, info.num_subcores, info.num_lanes
    nw = nc * ns
    assert seq % nw == 0 and dim % lanes == 0
    ppw = seq // nw
    n_chunks = dim // lanes
    pairs = lanes // 2
    rho = math.pow(10000.0, -2.0 / dim)

    mesh = plsc.VectorSubcoreMesh(core_axis_name="c", subcore_axis_name="s")

    @functools.partial(
        pl.kernel,
        mesh=mesh,
        out_type=jax.ShapeDtypeStruct((batch, seq, dim), jnp.float32),
        scratch_types=[
            pltpu.VMEM((batch, ppw), jnp.int32),
            pltpu.VMEM((ppw, dim), jnp.float32),
            pltpu.VMEM((2, ppw // 2, dim), jnp.float32),
            pltpu.SemaphoreType.DMA,
            pltpu.SemaphoreType.DMA,
        ],
    )
    def emb(idx_hbm, table_hbm, out_hbm, idx_v, pe_v, rows_v, gsem, ssem):
        wid = lax.axis_index("s") * nc + lax.axis_index("c")
        pos0 = wid * ppw

        pltpu.sync_copy(idx_hbm.at[:, pl.ds(pos0, ppw)], idx_v)
        gathers = [None] * (2 * batch)
        gathers[0] = pltpu.async_copy(
            table_hbm.at[idx_v.at[0, pl.ds(0, ppw // 2)]], rows_v.at[0], gsem
        )

        iota = lax.iota(jnp.int32, lanes)
        j = iota >> 1
        powvec = jnp.float32(rho ** (pairs - 1))
        for jj in range(pairs - 2, -1, -1):
            powvec = jnp.where(j == jj, jnp.float32(rho ** jj), powvec)
        sign = (1 - 2 * (iota & 1)).astype(jnp.float32)

        def chunk(c):
            return pl.ds(c * lanes, lanes)

        n_chains = 4
        x0 = jnp.where((iota & 1) == 1, jnp.float32(1.0), jnp.float32(0.0))
        y0 = 1.0 - x0
        ff_bits = max(1, (nw - 1).bit_length())
        rho_chunk = jnp.float32(rho ** pairs)

        def chunk_body(c, theta):
            t2 = theta * theta
            s = jnp.float32(1.0 / 362880)
            for coef in (-1.0 / 5040, 1.0 / 120, -1.0 / 6, 1.0):
                s = s * t2 + jnp.float32(coef)
            sinv = theta * s
            v = jnp.float32(1.0 / 3628800)
            for coef in (-1.0 / 40320, 1.0 / 720, -1.0 / 24, 0.5):
                v = v * t2 + jnp.float32(coef)
            v = v * t2
            cc1 = 1.0 - v
            ss1 = sinv * sign

            dbl = {}
            ds, dv = ss1, v
            for i in range(1, (ppw * (2 ** (ff_bits - 1))).bit_length()):
                ds, dv = 2.0 * ds - 2.0 * (ds * dv), 2.0 * (ds * ds)
                dbl[1 << i] = (ds, dv)

            x, y = x0, y0
            for j in range(ff_bits):
                ssj, vvj = dbl[ppw << j]
                xr = x - x * vvj + y * ssj
                yr = y - y * vvj - x * ssj
                bit = ((wid >> j) & 1) == 1
                x = jnp.where(bit, xr, x)
                y = jnp.where(bit, yr, y)

            hx, hy = [x], [y]
            for _ in range(n_chains - 1):
                px, py = hx[-1], hy[-1]
                hx.append(px * cc1 + py * ss1)
                hy.append(py * cc1 - px * ss1)

            ss4, vv4 = dbl[n_chains]
            cc4 = 1.0 - vv4

            def gen(t, carry, cc4=cc4, ss4=ss4):
                xs, ys = carry
                base = t * n_chains
                col = pl.ds(c * lanes, lanes)
                nxs, nys = [], []
                for k in range(n_chains):
                    pe_v[base + k, col] = xs[k]
                    nxs.append(xs[k] * cc4 + ys[k] * ss4)
                    nys.append(ys[k] * cc4 - xs[k] * ss4)
                return (tuple(nxs), tuple(nys))

            lax.fori_loop(0, ppw // n_chains, gen, (tuple(hx), tuple(hy)))
            return theta * rho_chunk

        lax.fori_loop(0, n_chunks, chunk_body, powvec)

        hpw = ppw // 2
        n_stages = 2 * batch
        stores = [None] * n_stages
        for s in range(n_stages):
            buf = s % 2
            if s + 1 < n_stages:
                if stores[s - 1] is not None:
                    stores[s - 1].wait()
                bn, hn = divmod(s + 1, 2)
                gathers[s + 1] = pltpu.async_copy(
                    table_hbm.at[idx_v.at[bn, pl.ds(hn * hpw, hpw)]],
                    rows_v.at[(s + 1) % 2],
                    gsem,
                )
            gathers[s].wait()
            b, h = divmod(s, 2)

            def row_add(r, carry, buf=buf, h=h):
                for c in range(n_chunks):
                    plsc.addupdate(
                        rows_v.at[buf, r, chunk(c)], pe_v[h * hpw + r, chunk(c)]
                    )
                return carry

            lax.fori_loop(0, hpw, row_add, 0)

            stores[s] = pltpu.async_copy(
                rows_v.at[buf], out_hbm.at[b, pl.ds(pos0 + h * hpw, hpw)], ssem
            )
        stores[n_stages - 2].wait()
        stores[n_stages - 1].wait()

    return emb


def kernel(input_ids, table):
    batch, seq = input_ids.shape
    dim = table.shape[1]
    idx = input_ids.astype(jnp.int32)
    return _build(batch, seq, dim)(idx, table)


# speedup vs baseline: 1.0760x; 1.0760x over previous
import functools
import math

import jax
import jax.numpy as jnp
from jax import lax
from jax.experimental import pallas as pl
from jax.experimental.pallas import tpu as pltpu
from jax.experimental.pallas import tpu_sc as plsc

N_VOCAB = 100000
MAX_LENGTH = 4096
OUT_DIM = 128


@functools.cache
def _build(batch, seq, dim):
    info = plsc.get_sparse_core_info()
    nc, ns, lanes = info.num_cores, info.num_subcores, info.num_lanes
    nw = nc * ns
    assert seq % nw == 0 and dim % lanes == 0
    ppw = seq // nw
    n_chunks = dim // lanes
    pairs = lanes // 2
    rho = math.pow(10000.0, -2.0 / dim)

    mesh = plsc.VectorSubcoreMesh(core_axis_name="c", subcore_axis_name="s")

    @functools.partial(
        pl.kernel,
        mesh=mesh,
        out_type=jax.ShapeDtypeStruct((batch, seq, dim), jnp.float32),
        scratch_types=[
            pltpu.VMEM((batch, ppw), jnp.int32),
            pltpu.VMEM((ppw, dim), jnp.float32),
            pltpu.VMEM((2, ppw, dim), jnp.float32),
            pltpu.SemaphoreType.DMA,
            pltpu.SemaphoreType.DMA,
        ],
    )
    def emb(idx_hbm, table_hbm, out_hbm, idx_v, pe_v, rows_v, gsem, ssem):
        wid = lax.axis_index("s") * nc + lax.axis_index("c")
        pos0 = wid * ppw

        pltpu.sync_copy(idx_hbm.at[:, pl.ds(pos0, ppw)], idx_v)
        gathers = [None] * batch
        gathers[0] = pltpu.async_copy(
            table_hbm.at[idx_v.at[0]], rows_v.at[0], gsem
        )

        iota = lax.iota(jnp.int32, lanes)
        j = iota >> 1
        powvec = jnp.float32(rho ** (pairs - 1))
        for jj in range(pairs - 2, -1, -1):
            powvec = jnp.where(j == jj, jnp.float32(rho ** jj), powvec)
        sign = (1 - 2 * (iota & 1)).astype(jnp.float32)

        def chunk(c):
            return pl.ds(c * lanes, lanes)

        n_chains = 4
        x0 = jnp.where((iota & 1) == 1, jnp.float32(1.0), jnp.float32(0.0))
        y0 = 1.0 - x0
        ff_bits = max(1, (nw - 1).bit_length())
        rho_chunk = jnp.float32(rho ** pairs)

        def chunk_body(c, theta):
            t2 = theta * theta
            s = jnp.float32(1.0 / 362880)
            for coef in (-1.0 / 5040, 1.0 / 120, -1.0 / 6, 1.0):
                s = s * t2 + jnp.float32(coef)
            sinv = theta * s
            v = jnp.float32(1.0 / 3628800)
            for coef in (-1.0 / 40320, 1.0 / 720, -1.0 / 24, 0.5):
                v = v * t2 + jnp.float32(coef)
            v = v * t2
            cc1 = 1.0 - v
            ss1 = sinv * sign

            dbl = {}
            ds, dv = ss1, v
            for i in range(1, (ppw * (2 ** (ff_bits - 1))).bit_length()):
                ds, dv = 2.0 * ds - 2.0 * (ds * dv), 2.0 * (ds * ds)
                dbl[1 << i] = (ds, dv)

            x, y = x0, y0
            for j in range(ff_bits):
                ssj, vvj = dbl[ppw << j]
                xr = x - x * vvj + y * ssj
                yr = y - y * vvj - x * ssj
                bit = ((wid >> j) & 1) == 1
                x = jnp.where(bit, xr, x)
                y = jnp.where(bit, yr, y)

            hx, hy = [x], [y]
            for _ in range(n_chains - 1):
                px, py = hx[-1], hy[-1]
                hx.append(px * cc1 + py * ss1)
                hy.append(py * cc1 - px * ss1)

            ss4, vv4 = dbl[n_chains]
            cc4 = 1.0 - vv4

            def gen(t, carry, cc4=cc4, ss4=ss4):
                xs, ys = carry
                base = t * n_chains
                col = pl.ds(c * lanes, lanes)
                nxs, nys = [], []
                for k in range(n_chains):
                    pe_v[base + k, col] = xs[k]
                    nxs.append(xs[k] * cc4 + ys[k] * ss4)
                    nys.append(ys[k] * cc4 - xs[k] * ss4)
                return (tuple(nxs), tuple(nys))

            lax.fori_loop(0, ppw // n_chains, gen, (tuple(hx), tuple(hy)))
            return theta * rho_chunk

        lax.fori_loop(0, n_chunks, chunk_body, powvec)

        stores = [None] * batch
        for b in range(batch):
            buf = b % 2
            if b + 1 < batch:
                if stores[b - 1] is not None:
                    stores[b - 1].wait()
                gathers[b + 1] = pltpu.async_copy(
                    table_hbm.at[idx_v.at[b + 1]], rows_v.at[(b + 1) % 2], gsem
                )
            gathers[b].wait()

            def row_add(r, carry, buf=buf):
                for c in range(n_chunks):
                    plsc.addupdate(rows_v.at[buf, r, chunk(c)], pe_v[r, chunk(c)])
                return carry

            lax.fori_loop(0, ppw, row_add, 0)

            stores[b] = pltpu.async_copy(
                rows_v.at[buf], out_hbm.at[b, pl.ds(pos0, ppw)], ssem
            )
        stores[batch - 2].wait()
        stores[batch - 1].wait()

    return emb


def kernel(input_ids, table):
    batch, seq = input_ids.shape
    dim = table.shape[1]
    idx = input_ids.astype(jnp.int32)
    return _build(batch, seq, dim)(idx, table)


# speedup vs baseline: 1.0766x; 1.0006x over previous
import functools
import math

import jax
import jax.numpy as jnp
from jax import lax
from jax.experimental import pallas as pl
from jax.experimental.pallas import tpu as pltpu
from jax.experimental.pallas import tpu_sc as plsc

N_VOCAB = 100000
MAX_LENGTH = 4096
OUT_DIM = 128


@functools.cache
def _build(batch, seq, dim):
    info = plsc.get_sparse_core_info()
    nc, ns, lanes = info.num_cores, info.num_subcores, info.num_lanes
    nw = nc * ns
    assert seq % nw == 0 and dim % lanes == 0
    ppw = seq // nw
    n_chunks = dim // lanes
    pairs = lanes // 2
    rho = math.pow(10000.0, -2.0 / dim)

    mesh = plsc.VectorSubcoreMesh(core_axis_name="c", subcore_axis_name="s")

    @functools.partial(
        pl.kernel,
        mesh=mesh,
        out_type=jax.ShapeDtypeStruct((batch, seq, dim), jnp.float32),
        scratch_types=[
            pltpu.VMEM((batch, ppw), jnp.int32),
            pltpu.VMEM((ppw, dim), jnp.float32),
            pltpu.VMEM((2, ppw, dim), jnp.float32),
            pltpu.SemaphoreType.DMA,
            pltpu.SemaphoreType.DMA,
        ],
    )
    def emb(idx_hbm, table_hbm, out_hbm, idx_v, pe_v, rows_v, gsem, ssem):
        wid = lax.axis_index("s") * nc + lax.axis_index("c")
        pos0 = wid * ppw

        pltpu.sync_copy(idx_hbm.at[:, pl.ds(pos0, ppw)], idx_v)
        gathers = [None] * batch
        for b in range(min(2, batch)):
            gathers[b] = pltpu.async_copy(
                table_hbm.at[idx_v.at[b]], rows_v.at[b], gsem
            )

        iota = lax.iota(jnp.int32, lanes)
        j = iota >> 1
        powvec = jnp.float32(rho ** (pairs - 1))
        for jj in range(pairs - 2, -1, -1):
            powvec = jnp.where(j == jj, jnp.float32(rho ** jj), powvec)
        sign = (1 - 2 * (iota & 1)).astype(jnp.float32)

        def chunk(c):
            return pl.ds(c * lanes, lanes)

        n_chains = 4
        x0 = jnp.where((iota & 1) == 1, jnp.float32(1.0), jnp.float32(0.0))
        y0 = 1.0 - x0
        ff_bits = max(1, (nw - 1).bit_length())
        rho_chunk = jnp.float32(rho ** pairs)

        def chunk_body(c, theta):
            t2 = theta * theta
            s = jnp.float32(1.0 / 362880)
            for coef in (-1.0 / 5040, 1.0 / 120, -1.0 / 6, 1.0):
                s = s * t2 + jnp.float32(coef)
            sinv = theta * s
            v = jnp.float32(1.0 / 3628800)
            for coef in (-1.0 / 40320, 1.0 / 720, -1.0 / 24, 0.5):
                v = v * t2 + jnp.float32(coef)
            v = v * t2
            cc1 = 1.0 - v
            ss1 = sinv * sign

            dbl = {}
            ds, dv = ss1, v
            for i in range(1, (ppw * (2 ** (ff_bits - 1))).bit_length()):
                ds, dv = 2.0 * ds - 2.0 * (ds * dv), 2.0 * (ds * ds)
                dbl[1 << i] = (ds, dv)

            x, y = x0, y0
            for j in range(ff_bits):
                ssj, vvj = dbl[ppw << j]
                xr = x - x * vvj + y * ssj
                yr = y - y * vvj - x * ssj
                bit = ((wid >> j) & 1) == 1
                x = jnp.where(bit, xr, x)
                y = jnp.where(bit, yr, y)

            hx, hy = [x], [y]
            for _ in range(n_chains - 1):
                px, py = hx[-1], hy[-1]
                hx.append(px * cc1 + py * ss1)
                hy.append(py * cc1 - px * ss1)

            ss4, vv4 = dbl[n_chains]
            cc4 = 1.0 - vv4

            def gen(t, carry, cc4=cc4, ss4=ss4):
                xs, ys = carry
                base = t * n_chains
                col = pl.ds(c * lanes, lanes)
                nxs, nys = [], []
                for k in range(n_chains):
                    pe_v[base + k, col] = xs[k]
                    nxs.append(xs[k] * cc4 + ys[k] * ss4)
                    nys.append(ys[k] * cc4 - xs[k] * ss4)
                return (tuple(nxs), tuple(nys))

            lax.fori_loop(0, ppw // n_chains, gen, (tuple(hx), tuple(hy)))
            return theta * rho_chunk

        lax.fori_loop(0, n_chunks, chunk_body, powvec)

        stores = [None] * batch
        for b in range(batch):
            buf = b % 2
            if b + 2 < batch:
                pass
            if 1 <= b and b + 1 < batch:
                stores[b - 1].wait()
                gathers[b + 1] = pltpu.async_copy(
                    table_hbm.at[idx_v.at[b + 1]], rows_v.at[(b + 1) % 2], gsem
                )
            gathers[b].wait()

            def row_add(r, carry, buf=buf):
                for c in range(n_chunks):
                    plsc.addupdate(rows_v.at[buf, r, chunk(c)], pe_v[r, chunk(c)])
                return carry

            lax.fori_loop(0, ppw, row_add, 0)

            stores[b] = pltpu.async_copy(
                rows_v.at[buf], out_hbm.at[b, pl.ds(pos0, ppw)], ssem
            )
        stores[batch - 2].wait()
        stores[batch - 1].wait()

    return emb


def kernel(input_ids, table):
    batch, seq = input_ids.shape
    dim = table.shape[1]
    idx = input_ids.astype(jnp.int32)
    return _build(batch, seq, dim)(idx, table)
